# baseline (speedup 1.0000x reference)
import jax
import jax.numpy as jnp
from jax.experimental import pallas as pl
from jax.experimental.pallas import tpu as pltpu

VOCAB = 100000
EMBED = 10
CTX = 2
HIDDEN = 128
T = 8192
K = (VOCAB + T - 1) // T
PAD = K * T


def _dense_body(embeds_ref, w1_ref, b1_ref, w2_ref, b2_ref, out_ref,
                m_ref, s_ref, h_ref):
    i = pl.program_id(0)

    @pl.when(i == 0)
    def _init():
        e = embeds_ref[...]
        h = jnp.dot(e, w1_ref[...], preferred_element_type=jnp.float32)
        h_ref[...] = jnp.maximum(h + b1_ref[...], 0.0)
        m_ref[...] = jnp.full((1, 1), -jnp.inf, jnp.float32)
        s_ref[...] = jnp.zeros((1, 1), jnp.float32)

    h = h_ref[...]
    logits = jnp.dot(h, w2_ref[...], preferred_element_type=jnp.float32)
    logits = logits + b2_ref[...]
    col = i * T + jax.lax.broadcasted_iota(jnp.int32, (1, T), 1)
    masked = jnp.where(col < VOCAB, logits, -jnp.inf)
    out_ref[0:1, pl.ds(i * T, T)] = masked

    m_old = m_ref[...]
    m_new = jnp.maximum(m_old, jnp.max(masked, keepdims=True))
    s_ref[...] = (s_ref[...] * jnp.exp(m_old - m_new)
                  + jnp.sum(jnp.exp(masked - m_new), keepdims=True))
    m_ref[...] = m_new

    @pl.when(i == K - 1)
    def _finish():
        logz = m_ref[...] + jnp.log(s_ref[...])
        out_ref[...] = out_ref[...] - logz


def _dense(embeds, W1, b1, W2, b2, interpret=False):
    out = pl.pallas_call(
        _dense_body,
        grid=(K,),
        in_specs=[
            pl.BlockSpec((1, CTX * EMBED), lambda i: (0, 0)),
            pl.BlockSpec((CTX * EMBED, HIDDEN), lambda i: (0, 0)),
            pl.BlockSpec((1, HIDDEN), lambda i: (0, 0)),
            pl.BlockSpec((HIDDEN, T), lambda i: (0, i)),
            pl.BlockSpec((1, T), lambda i: (0, i)),
        ],
        out_specs=pl.BlockSpec((1, PAD), lambda i: (0, 0)),
        out_shape=jax.ShapeDtypeStruct((1, PAD), jnp.float32),
        scratch_shapes=[
            pltpu.VMEM((1, 1), jnp.float32),
            pltpu.VMEM((1, 1), jnp.float32),
            pltpu.VMEM((1, HIDDEN), jnp.float32),
        ],
        interpret=interpret,
    )(embeds, W1, b1.reshape(1, HIDDEN), W2, b2.reshape(1, VOCAB))
    return out[:, :VOCAB]


def kernel(x, emb, W1, b1, W2, b2):
    embeds = jnp.take(emb, x, axis=0).reshape(1, CTX * EMBED)
    return _dense(embeds, W1, b1, W2, b2)


# speedup vs baseline: 1.0233x; 1.0233x over previous
import jax
import jax.numpy as jnp
from jax.experimental import pallas as pl
from jax.experimental.pallas import tpu as pltpu

VOCAB = 100000
EMBED = 10
CTX = 2
HIDDEN = 128
T = 16384
K = (VOCAB + T - 1) // T
PAD = K * T


def _dense_body(embeds_ref, w1_ref, b1_ref, w2_ref, b2_ref, out_ref,
                m_ref, s_ref, h_ref):
    i = pl.program_id(0)

    @pl.when(i == 0)
    def _init():
        e = embeds_ref[...]
        h = jnp.dot(e, w1_ref[...], preferred_element_type=jnp.float32)
        h_ref[...] = jnp.maximum(h + b1_ref[...], 0.0)
        m_ref[...] = jnp.full((1, 1), -jnp.inf, jnp.float32)
        s_ref[...] = jnp.zeros((1, 1), jnp.float32)

    h = h_ref[...]
    logits = jnp.dot(h, w2_ref[...], preferred_element_type=jnp.float32)
    logits = logits + b2_ref[...]
    col = i * T + jax.lax.broadcasted_iota(jnp.int32, (1, T), 1)
    masked = jnp.where(col < VOCAB, logits, -jnp.inf)
    out_ref[0:1, pl.ds(i * T, T)] = masked

    m_old = m_ref[...]
    m_new = jnp.maximum(m_old, jnp.max(masked, keepdims=True))
    s_ref[...] = (s_ref[...] * jnp.exp(m_old - m_new)
                  + jnp.sum(jnp.exp(masked - m_new), keepdims=True))
    m_ref[...] = m_new

    @pl.when(i == K - 1)
    def _finish():
        logz = m_ref[...] + jnp.log(s_ref[...])
        out_ref[...] = out_ref[...] - logz


def _dense(embeds, W1, b1, W2, b2, interpret=False):
    out = pl.pallas_call(
        _dense_body,
        grid=(K,),
        in_specs=[
            pl.BlockSpec((1, CTX * EMBED), lambda i: (0, 0)),
            pl.BlockSpec((CTX * EMBED, HIDDEN), lambda i: (0, 0)),
            pl.BlockSpec((1, HIDDEN), lambda i: (0, 0)),
            pl.BlockSpec((HIDDEN, T), lambda i: (0, i)),
            pl.BlockSpec((1, T), lambda i: (0, i)),
        ],
        out_specs=pl.BlockSpec((1, PAD), lambda i: (0, 0)),
        out_shape=jax.ShapeDtypeStruct((1, PAD), jnp.float32),
        scratch_shapes=[
            pltpu.VMEM((1, 1), jnp.float32),
            pltpu.VMEM((1, 1), jnp.float32),
            pltpu.VMEM((1, HIDDEN), jnp.float32),
        ],
        interpret=interpret,
    )(embeds, W1, b1.reshape(1, HIDDEN), W2, b2.reshape(1, VOCAB))
    return out[:, :VOCAB]


def kernel(x, emb, W1, b1, W2, b2):
    embeds = jnp.take(emb, x, axis=0).reshape(1, CTX * EMBED)
    return _dense(embeds, W1, b1, W2, b2)
